# Initial kernel scaffold; baseline (speedup 1.0000x reference)
import jax
import jax.numpy as jnp
from jax.experimental import pallas as pl


def kernel(xs, pos_enc, gat_lin, gat_src, gat_dst, gat_bias, lin_final):
    raise NotImplementedError("write your pallas kernel here")


# speedup vs baseline: 1821.3738x; 1821.3738x over previous
import jax
import jax.numpy as jnp
from jax.experimental import pallas as pl

BS, NUM_ROWS, NUM_XS, ENC_DIM = 4, 128, 64, 15
HEADS, OUT_PER_HEAD, NUM_LAYERS = 4, 4, 2
IN_DIM = 1 + ENC_DIM
HID = HEADS * OUT_PER_HEAD
NUM_CLASSES = 2

ROW_TILE = 32


def _gat_kernel(x_ref, lin_ref, msrc_ref, mdst_ref, bias_ref, linf_ref, out_ref):
    x3 = x_ref[0]
    for l in range(NUM_LAYERS):
        lin = lin_ref[0, l]
        xp3 = jax.lax.dot_general(
            x3, lin, (((2,), (1,)), ((), ())),
            preferred_element_type=jnp.float32)
        a_src = jax.lax.dot_general(
            xp3, msrc_ref[0, l], (((2,), (0,)), ((), ())),
            preferred_element_type=jnp.float32)
        a_dst = jax.lax.dot_general(
            xp3, mdst_ref[0, l], (((2,), (0,)), ((), ())),
            preferred_element_type=jnp.float32)
        aggs = []
        for h in range(HEADS):
            d_h = a_dst[:, :, h:h + 1]
            s_h = jnp.swapaxes(a_src[:, :, h:h + 1], 1, 2)
            logits = d_h + s_h
            logits = jnp.where(logits >= 0, logits, 0.2 * logits)
            m = jnp.max(logits, axis=2, keepdims=True)
            e = jnp.exp(logits - m)
            den = jnp.sum(e, axis=2, keepdims=True) + 1e-16
            attn = e / den
            xp_h = xp3[:, :, h * OUT_PER_HEAD:(h + 1) * OUT_PER_HEAD]
            agg_h = jax.lax.dot_general(
                attn, xp_h, (((2,), (1,)), ((0,), (0,))),
                preferred_element_type=jnp.float32)
            aggs.append(agg_h)
        x3 = jnp.concatenate(aggs, axis=2) + bias_ref[0, l]
    xsum = jnp.sum(x3, axis=1)
    out_ref[0] = jax.lax.dot_general(
        xsum, linf_ref[0], (((1,), (0,)), ((), ())),
        preferred_element_type=jnp.float32)


def kernel(xs, pos_enc, gat_lin, gat_src, gat_dst, gat_bias, lin_final):
    bs, num_rows, num_xs = xs.shape
    pe = jnp.broadcast_to(pos_enc[:, None], (bs, num_rows, num_xs, ENC_DIM))
    x_all = jnp.concatenate([xs[..., None], pe], axis=-1)

    eye = jnp.eye(HEADS, dtype=xs.dtype)
    msrc = (gat_src[:, :, 0, :, :, None] * eye[:, None, :]).reshape(
        bs, NUM_LAYERS, HID, HEADS)
    mdst = (gat_dst[:, :, 0, :, :, None] * eye[:, None, :]).reshape(
        bs, NUM_LAYERS, HID, HEADS)
    bias = gat_bias.reshape(bs, NUM_LAYERS, 1, HID)
    linf = jnp.swapaxes(lin_final, 1, 2)

    r = ROW_TILE
    grid = (bs, num_rows // r)
    out = pl.pallas_call(
        _gat_kernel,
        grid=grid,
        in_specs=[
            pl.BlockSpec((1, r, num_xs, IN_DIM), lambda b, i: (b, i, 0, 0)),
            pl.BlockSpec((1, NUM_LAYERS, HID, IN_DIM), lambda b, i: (b, 0, 0, 0)),
            pl.BlockSpec((1, NUM_LAYERS, HID, HEADS), lambda b, i: (b, 0, 0, 0)),
            pl.BlockSpec((1, NUM_LAYERS, HID, HEADS), lambda b, i: (b, 0, 0, 0)),
            pl.BlockSpec((1, NUM_LAYERS, 1, HID), lambda b, i: (b, 0, 0, 0)),
            pl.BlockSpec((1, HID, NUM_CLASSES), lambda b, i: (b, 0, 0)),
        ],
        out_specs=pl.BlockSpec((1, r, NUM_CLASSES), lambda b, i: (b, i, 0)),
        out_shape=jax.ShapeDtypeStruct((bs, num_rows, NUM_CLASSES), xs.dtype),
    )(x_all, gat_lin, msrc, mdst, bias, linf)
    return out


# speedup vs baseline: 5453.3239x; 2.9941x over previous
import jax
import jax.numpy as jnp
from jax.experimental import pallas as pl

BS, NUM_ROWS, NUM_XS, ENC_DIM = 4, 128, 64, 15
HEADS, OUT_PER_HEAD, NUM_LAYERS = 4, 4, 2
IN_DIM = 1 + ENC_DIM
HID = HEADS * OUT_PER_HEAD
NUM_CLASSES = 2
LANES = HEADS * NUM_XS

ROW_TILE = 32


def _gat_kernel(x_ref, lin_ref, msrc_ref, mdst_ref, bias_ref, linf_ref, out_ref):
    f32 = jnp.float32
    i32 = jnp.int32
    lane_src = jax.lax.broadcasted_iota(i32, (NUM_XS, LANES), 1) % NUM_XS
    node_idx = jax.lax.broadcasted_iota(i32, (NUM_XS, LANES), 0)
    dmask = (lane_src == node_idx).astype(f32)
    bsum = (jax.lax.broadcasted_iota(i32, (LANES, HEADS), 0) // NUM_XS ==
            jax.lax.broadcasted_iota(i32, (LANES, HEADS), 1)).astype(f32)
    e16 = (jax.lax.broadcasted_iota(i32, (HEADS, HID), 0) ==
           jax.lax.broadcasted_iota(i32, (HEADS, HID), 1) // OUT_PER_HEAD
           ).astype(f32)
    ch = jax.lax.broadcasted_iota(i32, (1, 1, HID), 2) // OUT_PER_HEAD

    x3 = x_ref[0]
    for l in range(NUM_LAYERS):
        lin = lin_ref[0, l]
        xp3 = jax.lax.dot_general(
            x3, lin, (((2,), (1,)), ((), ())),
            preferred_element_type=f32)
        a_dst_big = jax.lax.dot_general(
            xp3, mdst_ref[0, l], (((2,), (0,)), ((), ())),
            preferred_element_type=f32)
        tmp_src = jax.lax.dot_general(
            xp3, msrc_ref[0, l], (((2,), (0,)), ((), ())),
            preferred_element_type=f32)
        v = jnp.sum(tmp_src * dmask, axis=1, keepdims=True)
        logits = a_dst_big + v
        logits = jnp.where(logits >= 0, logits, 0.2 * logits)
        m = jnp.max(logits, axis=2, keepdims=True)
        e = jnp.exp(logits - m)
        den = jax.lax.dot_general(
            e, bsum, (((2,), (0,)), ((), ())),
            preferred_element_type=f32)
        rcp = (1.0 / (den + 1e-16)) @ e16
        xstack = jnp.concatenate(
            [xp3 * (ch == h).astype(f32) for h in range(HEADS)],
            axis=1)
        agg = jax.lax.dot_general(
            e, xstack, (((2,), (1,)), ((0,), (0,))),
            preferred_element_type=f32)
        x3 = agg * rcp + bias_ref[0, l]
    xsum = jnp.sum(x3, axis=1)
    out_ref[0] = jax.lax.dot_general(
        xsum, linf_ref[0], (((1,), (0,)), ((), ())),
        preferred_element_type=f32)


def kernel(xs, pos_enc, gat_lin, gat_src, gat_dst, gat_bias, lin_final):
    bs, num_rows, num_xs = xs.shape
    pe = jnp.broadcast_to(pos_enc[:, None], (bs, num_rows, num_xs, ENC_DIM))
    x_all = jnp.concatenate([xs[..., None], pe], axis=-1)

    eye = jnp.eye(HEADS, dtype=xs.dtype)
    msrc = (gat_src[:, :, 0, :, :, None] * eye[:, None, :]).reshape(
        bs, NUM_LAYERS, HID, HEADS)
    mdst = (gat_dst[:, :, 0, :, :, None] * eye[:, None, :]).reshape(
        bs, NUM_LAYERS, HID, HEADS)
    msrc_e = jnp.repeat(msrc, NUM_XS, axis=-1)
    mdst_e = jnp.repeat(mdst, NUM_XS, axis=-1)
    bias = gat_bias.reshape(bs, NUM_LAYERS, 1, HID)
    linf = jnp.swapaxes(lin_final, 1, 2)

    r = ROW_TILE
    grid = (bs, num_rows // r)
    out = pl.pallas_call(
        _gat_kernel,
        grid=grid,
        in_specs=[
            pl.BlockSpec((1, r, num_xs, IN_DIM), lambda b, i: (b, i, 0, 0)),
            pl.BlockSpec((1, NUM_LAYERS, HID, IN_DIM), lambda b, i: (b, 0, 0, 0)),
            pl.BlockSpec((1, NUM_LAYERS, HID, LANES), lambda b, i: (b, 0, 0, 0)),
            pl.BlockSpec((1, NUM_LAYERS, HID, LANES), lambda b, i: (b, 0, 0, 0)),
            pl.BlockSpec((1, NUM_LAYERS, 1, HID), lambda b, i: (b, 0, 0, 0)),
            pl.BlockSpec((1, HID, NUM_CLASSES), lambda b, i: (b, 0, 0)),
        ],
        out_specs=pl.BlockSpec((1, r, NUM_CLASSES), lambda b, i: (b, i, 0)),
        out_shape=jax.ShapeDtypeStruct((bs, num_rows, NUM_CLASSES), xs.dtype),
    )(x_all, gat_lin, msrc_e, mdst_e, bias, linf)
    return out
